# Initial kernel scaffold; baseline (speedup 1.0000x reference)
import jax
import jax.numpy as jnp
from jax.experimental import pallas as pl


def kernel(x, Wg, w1, w2, w3, ws1, ws2, ws3):
    raise NotImplementedError("write your pallas kernel here")


# speedup vs baseline: 1.1084x; 1.1084x over previous
import functools

import jax
import jax.numpy as jnp
from jax import lax
from jax.experimental import pallas as pl
from jax.experimental.pallas import tpu as pltpu

T = 2048
DIM = 1024
E = 8
INTER = 512
SI = 1024
BS = 256
NB = T * 2 // BS + E
NPAD = NB * BS
WL = 16

_INTERPRET = False


def _routing_body(x_ref, wg_ref, pos0_ref, pos1_ref, w0_ref, w1_ref, be_ref):
    xf = x_ref[...]
    wg = wg_ref[...]
    s = lax.dot_general(xf, wg, (((1,), (1,)), ((), ())),
                        preferred_element_type=jnp.float32)
    m = jnp.max(s, axis=1, keepdims=True)
    ex = jnp.exp(s - m)
    p = ex / jnp.sum(ex, axis=1, keepdims=True)

    idx = lax.broadcasted_iota(jnp.int32, (T, E), 1)
    m0 = jnp.max(p, axis=1, keepdims=True)
    c0 = jnp.min(jnp.where(p == m0, idx, E), axis=1, keepdims=True)
    oh0 = idx == c0
    p2 = jnp.where(oh0, -jnp.inf, p)
    m1 = jnp.max(p2, axis=1, keepdims=True)
    c1 = jnp.min(jnp.where(p2 == m1, idx, E), axis=1, keepdims=True)
    oh1 = idx == c1

    C = oh0.astype(jnp.int32) + oh1.astype(jnp.int32)
    inc = C
    sh = 1
    while sh < T:
        inc = inc + jnp.concatenate(
            [jnp.zeros((sh, E), jnp.int32), inc[: T - sh]], axis=0)
        sh *= 2
    exc = inc - C
    counts = inc[T - 1: T]
    padded = ((counts + (BS - 1)) // BS) * BS
    pc = padded
    sh = 1
    while sh < E:
        pc = pc + jnp.concatenate(
            [jnp.zeros((1, sh), jnp.int32), pc[:, : E - sh]], axis=1)
        sh *= 2
    off = pc - padded
    start = exc + off
    pos0 = jnp.sum(jnp.where(oh0, start, 0), axis=1, keepdims=True)
    pos1 = jnp.sum(jnp.where(oh1, start, 0), axis=1, keepdims=True)
    pos0_ref[...] = pos0
    pos1_ref[...] = pos1
    w0_ref[...] = jnp.broadcast_to(m0, (T, WL))
    w1_ref[...] = jnp.broadcast_to(m1, (T, WL))

    ends = jnp.broadcast_to(pc, (NB, E))
    bstart = lax.broadcasted_iota(jnp.int32, (NB, E), 0) * BS
    be = jnp.sum((ends <= bstart).astype(jnp.int32), axis=1, keepdims=True)
    be_ref[...] = be


def _routing(x, Wg):
    return pl.pallas_call(
        _routing_body,
        out_shape=[
            jax.ShapeDtypeStruct((T, 1), jnp.int32),
            jax.ShapeDtypeStruct((T, 1), jnp.int32),
            jax.ShapeDtypeStruct((T, WL), jnp.float32),
            jax.ShapeDtypeStruct((T, WL), jnp.float32),
            jax.ShapeDtypeStruct((NB, 1), jnp.int32),
        ],
        interpret=_INTERPRET,
    )(x, Wg)


def _gmm_body(be_ref, xs_ref, ws_ref, w1_ref, w3_ref, w2_ref, out_ref):
    i = pl.program_id(0)

    @pl.when(be_ref[i] < E)
    def _():
        xb = xs_ref[...]
        h1 = lax.dot_general(xb, w1_ref[0], (((1,), (1,)), ((), ())),
                             preferred_element_type=jnp.float32)
        h3 = lax.dot_general(xb, w3_ref[0], (((1,), (1,)), ((), ())),
                             preferred_element_type=jnp.float32)
        h = h1 * jax.nn.sigmoid(h1) * h3
        ye = lax.dot_general(h, w2_ref[0], (((1,), (1,)), ((), ())),
                             preferred_element_type=jnp.float32)
        out_ref[...] = ye * ws_ref[:, 0:1]


def _gmm(be, Xs, Ws, w1, w3, w2):
    def wmap(i, be_ref):
        return (jnp.minimum(be_ref[i], E - 1), 0, 0)

    grid_spec = pltpu.PrefetchScalarGridSpec(
        num_scalar_prefetch=1,
        grid=(NB,),
        in_specs=[
            pl.BlockSpec((BS, DIM), lambda i, be_ref: (i, 0)),
            pl.BlockSpec((BS, WL), lambda i, be_ref: (i, 0)),
            pl.BlockSpec((1, INTER, DIM), wmap),
            pl.BlockSpec((1, INTER, DIM), wmap),
            pl.BlockSpec((1, DIM, INTER), wmap),
        ],
        out_specs=pl.BlockSpec((BS, DIM), lambda i, be_ref: (i, 0)),
    )
    return pl.pallas_call(
        _gmm_body,
        grid_spec=grid_spec,
        out_shape=jax.ShapeDtypeStruct((NPAD, DIM), jnp.float32),
        interpret=_INTERPRET,
    )(be, Xs, Ws, w1, w3, w2)


def _shared_body(x_ref, ws1_ref, ws3_ref, ws2_ref, z_ref):
    xb = x_ref[...]
    h1 = lax.dot_general(xb, ws1_ref[...], (((1,), (1,)), ((), ())),
                         preferred_element_type=jnp.float32)
    h3 = lax.dot_general(xb, ws3_ref[...], (((1,), (1,)), ((), ())),
                         preferred_element_type=jnp.float32)
    h = h1 * jax.nn.sigmoid(h1) * h3
    z_ref[...] = lax.dot_general(h, ws2_ref[...], (((1,), (1,)), ((), ())),
                                 preferred_element_type=jnp.float32)


def _shared(x, ws1, ws3, ws2):
    return pl.pallas_call(
        _shared_body,
        grid=(T // BS,),
        in_specs=[
            pl.BlockSpec((BS, DIM), lambda i: (i, 0)),
            pl.BlockSpec((SI, DIM), lambda i: (0, 0)),
            pl.BlockSpec((SI, DIM), lambda i: (0, 0)),
            pl.BlockSpec((DIM, SI), lambda i: (0, 0)),
        ],
        out_specs=pl.BlockSpec((BS, DIM), lambda i: (i, 0)),
        out_shape=jax.ShapeDtypeStruct((T, DIM), jnp.float32),
        interpret=_INTERPRET,
    )(x, ws1, ws3, ws2)


def kernel(x, Wg, w1, w2, w3, ws1, ws2, ws3):
    shape = x.shape
    xf = x.reshape(T, DIM)
    pos0, pos1, w0s, w1s, be = _routing(xf, Wg)
    p0 = pos0.reshape(T)
    p1 = pos1.reshape(T)
    be = be.reshape(NB)

    Xs = jnp.zeros((NPAD, DIM), jnp.float32).at[p0].set(xf).at[p1].set(xf)
    Ws = jnp.zeros((NPAD, WL), jnp.float32).at[p0].set(w0s).at[p1].set(w1s)

    Yw = _gmm(be, Xs, Ws, w1, w3, w2)
    z = _shared(xf, ws1, ws3, ws2)

    y = Yw[p0] + Yw[p1] + z
    return y.reshape(shape)


# speedup vs baseline: 1.3457x; 1.2141x over previous
import functools

import jax
import jax.numpy as jnp
from jax import lax
from jax.experimental import pallas as pl
from jax.experimental.pallas import tpu as pltpu
from jax.experimental.pallas import tpu_sc as plsc

T = 2048
DIM = 1024
E = 8
INTER = 512
SI = 1024
BS = 256
NB = T * 2 // BS + E
NPAD = NB * BS
WL = 128

_INTERPRET = False


def _routing_body(x_ref, wg_ref, pos0_ref, pos1_ref, w0_ref, w1_ref, be_ref):
    xf = x_ref[...]
    wg = wg_ref[...]
    s = lax.dot_general(xf, wg, (((1,), (1,)), ((), ())),
                        preferred_element_type=jnp.float32)
    m = jnp.max(s, axis=1, keepdims=True)
    ex = jnp.exp(s - m)
    p = ex / jnp.sum(ex, axis=1, keepdims=True)

    idx = lax.broadcasted_iota(jnp.int32, (T, E), 1)
    m0 = jnp.max(p, axis=1, keepdims=True)
    c0 = jnp.min(jnp.where(p == m0, idx, E), axis=1, keepdims=True)
    oh0 = idx == c0
    p2 = jnp.where(oh0, -jnp.inf, p)
    m1 = jnp.max(p2, axis=1, keepdims=True)
    c1 = jnp.min(jnp.where(p2 == m1, idx, E), axis=1, keepdims=True)
    oh1 = idx == c1

    C = oh0.astype(jnp.int32) + oh1.astype(jnp.int32)
    inc = C
    sh = 1
    while sh < T:
        inc = inc + jnp.concatenate(
            [jnp.zeros((sh, E), jnp.int32), inc[: T - sh]], axis=0)
        sh *= 2
    exc = inc - C
    counts = inc[T - 1: T]
    padded = ((counts + (BS - 1)) // BS) * BS
    pc = padded
    sh = 1
    while sh < E:
        pc = pc + jnp.concatenate(
            [jnp.zeros((1, sh), jnp.int32), pc[:, : E - sh]], axis=1)
        sh *= 2
    off = pc - padded
    start = exc + off
    pos0 = jnp.sum(jnp.where(oh0, start, 0), axis=1, keepdims=True)
    pos1 = jnp.sum(jnp.where(oh1, start, 0), axis=1, keepdims=True)
    pos0_ref[...] = pos0
    pos1_ref[...] = pos1
    w0_ref[...] = jnp.broadcast_to(m0, (T, WL))
    w1_ref[...] = jnp.broadcast_to(m1, (T, WL))

    ends = jnp.broadcast_to(pc, (NB, E))
    bstart = lax.broadcasted_iota(jnp.int32, (NB, E), 0) * BS
    be = jnp.sum((ends <= bstart).astype(jnp.int32), axis=1, keepdims=True)
    be_ref[...] = be


def _routing(x, Wg):
    return pl.pallas_call(
        _routing_body,
        out_shape=[
            jax.ShapeDtypeStruct((T, 1), jnp.int32),
            jax.ShapeDtypeStruct((T, 1), jnp.int32),
            jax.ShapeDtypeStruct((T, WL), jnp.float32),
            jax.ShapeDtypeStruct((T, WL), jnp.float32),
            jax.ShapeDtypeStruct((NB, 1), jnp.int32),
        ],
        interpret=_INTERPRET,
    )(x, Wg)


def _gmm_body(be_ref, xs_ref, ws_ref, w1_ref, w3_ref, w2_ref, out_ref):
    i = pl.program_id(0)

    @pl.when(be_ref[i] < E)
    def _():
        xb = xs_ref[...]
        h1 = lax.dot_general(xb, w1_ref[0], (((1,), (1,)), ((), ())),
                             preferred_element_type=jnp.float32)
        h3 = lax.dot_general(xb, w3_ref[0], (((1,), (1,)), ((), ())),
                             preferred_element_type=jnp.float32)
        h = h1 * jax.nn.sigmoid(h1) * h3
        ye = lax.dot_general(h, w2_ref[0], (((1,), (1,)), ((), ())),
                             preferred_element_type=jnp.float32)
        out_ref[...] = ye * ws_ref[:, 0:1]


def _gmm(be, Xs, Ws, w1, w3, w2):
    def wmap(i, be_ref):
        return (jnp.minimum(be_ref[i], E - 1), 0, 0)

    grid_spec = pltpu.PrefetchScalarGridSpec(
        num_scalar_prefetch=1,
        grid=(NB,),
        in_specs=[
            pl.BlockSpec((BS, DIM), lambda i, be_ref: (i, 0)),
            pl.BlockSpec((BS, WL), lambda i, be_ref: (i, 0)),
            pl.BlockSpec((1, INTER, DIM), wmap),
            pl.BlockSpec((1, INTER, DIM), wmap),
            pl.BlockSpec((1, DIM, INTER), wmap),
        ],
        out_specs=pl.BlockSpec((BS, DIM), lambda i, be_ref: (i, 0)),
    )
    return pl.pallas_call(
        _gmm_body,
        grid_spec=grid_spec,
        out_shape=jax.ShapeDtypeStruct((NPAD, DIM), jnp.float32),
        interpret=_INTERPRET,
    )(be, Xs, Ws, w1, w3, w2)


def _shared_body(x_ref, ws1_ref, ws3_ref, ws2_ref, z_ref):
    xb = x_ref[...]
    h1 = lax.dot_general(xb, ws1_ref[...], (((1,), (1,)), ((), ())),
                         preferred_element_type=jnp.float32)
    h3 = lax.dot_general(xb, ws3_ref[...], (((1,), (1,)), ((), ())),
                         preferred_element_type=jnp.float32)
    h = h1 * jax.nn.sigmoid(h1) * h3
    z_ref[...] = lax.dot_general(h, ws2_ref[...], (((1,), (1,)), ((), ())),
                                 preferred_element_type=jnp.float32)


def _shared(x, ws1, ws3, ws2):
    return pl.pallas_call(
        _shared_body,
        grid=(T // BS,),
        in_specs=[
            pl.BlockSpec((BS, DIM), lambda i: (i, 0)),
            pl.BlockSpec((SI, DIM), lambda i: (0, 0)),
            pl.BlockSpec((SI, DIM), lambda i: (0, 0)),
            pl.BlockSpec((DIM, SI), lambda i: (0, 0)),
        ],
        out_specs=pl.BlockSpec((BS, DIM), lambda i: (i, 0)),
        out_shape=jax.ShapeDtypeStruct((T, DIM), jnp.float32),
        interpret=_INTERPRET,
    )(x, ws1, ws3, ws2)


_NW = 32
_TPW = T // _NW


def _dispatch(xf, p0, p1, w0s, w1s):
    mesh = plsc.VectorSubcoreMesh(core_axis_name="c", subcore_axis_name="s")

    @functools.partial(
        pl.kernel, mesh=mesh,
        out_type=[
            jax.ShapeDtypeStruct((NPAD, DIM), jnp.float32),
            jax.ShapeDtypeStruct((NPAD, WL), jnp.float32),
        ],
        scratch_types=[
            pltpu.VMEM((_TPW,), jnp.int32),
            pltpu.VMEM((_TPW,), jnp.int32),
            pltpu.VMEM((_TPW, DIM), jnp.float32),
            pltpu.VMEM((_TPW, WL), jnp.float32),
            pltpu.VMEM((_TPW, WL), jnp.float32),
            pltpu.SemaphoreType.DMA,
        ],
    )
    def k(x_hbm, p0_hbm, p1_hbm, w0_hbm, w1_hbm, xs_hbm, ws_hbm,
          i0, i1, xa, wa0, wa1, sem):
        wid = lax.axis_index("s") * 2 + lax.axis_index("c")
        base = wid * _TPW
        pltpu.sync_copy(p0_hbm.at[pl.ds(base, _TPW)], i0)
        pltpu.sync_copy(p1_hbm.at[pl.ds(base, _TPW)], i1)
        pltpu.sync_copy(x_hbm.at[pl.ds(base, _TPW)], xa)
        pltpu.sync_copy(w0_hbm.at[pl.ds(base, _TPW)], wa0)
        pltpu.sync_copy(w1_hbm.at[pl.ds(base, _TPW)], wa1)
        pltpu.async_copy(xa, xs_hbm.at[i0], sem).wait()
        pltpu.async_copy(xa, xs_hbm.at[i1], sem).wait()
        pltpu.async_copy(wa0, ws_hbm.at[i0], sem).wait()
        pltpu.async_copy(wa1, ws_hbm.at[i1], sem).wait()

    return k(xf, p0, p1, w0s, w1s)


_CC = 16


def _combine(Yw, p0, p1, z):
    mesh = plsc.VectorSubcoreMesh(core_axis_name="c", subcore_axis_name="s")

    @functools.partial(
        pl.kernel, mesh=mesh,
        out_type=jax.ShapeDtypeStruct((T, DIM), jnp.float32),
        scratch_types=[
            pltpu.VMEM((_TPW,), jnp.int32),
            pltpu.VMEM((_TPW,), jnp.int32),
            pltpu.VMEM((_CC, DIM), jnp.float32),
            pltpu.VMEM((_CC, DIM), jnp.float32),
            pltpu.VMEM((_CC, DIM), jnp.float32),
            pltpu.SemaphoreType.DMA,
            pltpu.SemaphoreType.DMA,
        ],
    )
    def k(yw_hbm, p0_hbm, p1_hbm, z_hbm, y_hbm, i0, i1, a, b, zc, s0, s1):
        wid = lax.axis_index("s") * 2 + lax.axis_index("c")
        base = wid * _TPW
        pltpu.sync_copy(p0_hbm.at[pl.ds(base, _TPW)], i0)
        pltpu.sync_copy(p1_hbm.at[pl.ds(base, _TPW)], i1)
        for j in range(_TPW // _CC):
            ia = i0[pl.ds(j * _CC, _CC)]
            ib = i1[pl.ds(j * _CC, _CC)]
            ca = pltpu.async_copy(yw_hbm.at[ia], a, s0)
            cb = pltpu.async_copy(yw_hbm.at[ib], b, s1)
            pltpu.sync_copy(z_hbm.at[pl.ds(base + j * _CC, _CC)], zc)
            ca.wait()
            cb.wait()

            def body(t, _):
                def inner(c, _):
                    sl = pl.ds(c * 16, 16)
                    zc[t, sl] = zc[t, sl] + a[t, sl] + b[t, sl]
                    return 0
                return lax.fori_loop(0, DIM // 16, inner, 0)

            lax.fori_loop(0, _CC, body, 0)
            pltpu.sync_copy(zc, y_hbm.at[pl.ds(base + j * _CC, _CC)])

    return k(Yw, p0, p1, z)


def kernel(x, Wg, w1, w2, w3, ws1, ws2, ws3):
    shape = x.shape
    xf = x.reshape(T, DIM)
    pos0, pos1, w0s, w1s, be = _routing(xf, Wg)
    p0 = pos0.reshape(T)
    p1 = pos1.reshape(T)
    be = be.reshape(NB)

    Xs, Ws = _dispatch(xf, p0, p1, w0s, w1s)
    Yw = _gmm(be, Xs, Ws, w1, w3, w2)
    z = _shared(xf, ws1, ws3, ws2)
    y = _combine(Yw, p0, p1, z)
    return y.reshape(shape)


# speedup vs baseline: 1.4280x; 1.0612x over previous
import functools

import jax
import jax.numpy as jnp
from jax import lax
from jax.experimental import pallas as pl
from jax.experimental.pallas import tpu as pltpu
from jax.experimental.pallas import tpu_sc as plsc

T = 2048
DIM = 1024
E = 8
INTER = 512
SI = 1024
BS = 256
NB = T * 2 // BS + E
NPAD = NB * BS
WL = 128

_INTERPRET = False


def _routing_body(x_ref, wg_ref, pos0_ref, pos1_ref, w0_ref, w1_ref, be_ref):
    xf = x_ref[...]
    wg = wg_ref[...]
    s = lax.dot_general(xf, wg, (((1,), (1,)), ((), ())),
                        preferred_element_type=jnp.float32)
    m = jnp.max(s, axis=1, keepdims=True)
    ex = jnp.exp(s - m)
    p = ex / jnp.sum(ex, axis=1, keepdims=True)

    idx = lax.broadcasted_iota(jnp.int32, (T, E), 1)
    m0 = jnp.max(p, axis=1, keepdims=True)
    c0 = jnp.min(jnp.where(p == m0, idx, E), axis=1, keepdims=True)
    oh0 = idx == c0
    p2 = jnp.where(oh0, -jnp.inf, p)
    m1 = jnp.max(p2, axis=1, keepdims=True)
    c1 = jnp.min(jnp.where(p2 == m1, idx, E), axis=1, keepdims=True)
    oh1 = idx == c1

    C = oh0.astype(jnp.int32) + oh1.astype(jnp.int32)
    inc = C
    sh = 1
    while sh < T:
        inc = inc + jnp.concatenate(
            [jnp.zeros((sh, E), jnp.int32), inc[: T - sh]], axis=0)
        sh *= 2
    exc = inc - C
    counts = inc[T - 1: T]
    padded = ((counts + (BS - 1)) // BS) * BS
    pc = padded
    sh = 1
    while sh < E:
        pc = pc + jnp.concatenate(
            [jnp.zeros((1, sh), jnp.int32), pc[:, : E - sh]], axis=1)
        sh *= 2
    off = pc - padded
    start = exc + off
    pos0 = jnp.sum(jnp.where(oh0, start, 0), axis=1, keepdims=True)
    pos1 = jnp.sum(jnp.where(oh1, start, 0), axis=1, keepdims=True)
    pos0_ref[...] = pos0
    pos1_ref[...] = pos1
    w0_ref[...] = jnp.broadcast_to(m0, (T, WL))
    w1_ref[...] = jnp.broadcast_to(m1, (T, WL))

    ends = jnp.broadcast_to(pc, (NB, E))
    bstart = lax.broadcasted_iota(jnp.int32, (NB, E), 0) * BS
    be = jnp.sum((ends <= bstart).astype(jnp.int32), axis=1, keepdims=True)
    be_ref[...] = be


def _routing(x, Wg):
    return pl.pallas_call(
        _routing_body,
        out_shape=[
            jax.ShapeDtypeStruct((T, 1), jnp.int32),
            jax.ShapeDtypeStruct((T, 1), jnp.int32),
            jax.ShapeDtypeStruct((T, WL), jnp.float32),
            jax.ShapeDtypeStruct((T, WL), jnp.float32),
            jax.ShapeDtypeStruct((NB, 1), jnp.int32),
        ],
        interpret=_INTERPRET,
    )(x, Wg)


def _gmm_body(be_ref, xs_ref, ws_ref, w1_ref, w3_ref, w2_ref, out_ref):
    i = pl.program_id(0)

    @pl.when(be_ref[i] < E)
    def _():
        xb = xs_ref[...]
        h1 = lax.dot_general(xb, w1_ref[0], (((1,), (1,)), ((), ())),
                             preferred_element_type=jnp.float32)
        h3 = lax.dot_general(xb, w3_ref[0], (((1,), (1,)), ((), ())),
                             preferred_element_type=jnp.float32)
        h = h1 * jax.nn.sigmoid(h1) * h3
        ye = lax.dot_general(h, w2_ref[0], (((1,), (1,)), ((), ())),
                             preferred_element_type=jnp.float32)
        out_ref[...] = ye * ws_ref[:, 0:1]


def _gmm(be, Xs, Ws, w1, w3, w2):
    def wmap(i, be_ref):
        return (jnp.minimum(be_ref[i], E - 1), 0, 0)

    grid_spec = pltpu.PrefetchScalarGridSpec(
        num_scalar_prefetch=1,
        grid=(NB,),
        in_specs=[
            pl.BlockSpec((BS, DIM), lambda i, be_ref: (i, 0)),
            pl.BlockSpec((BS, WL), lambda i, be_ref: (i, 0)),
            pl.BlockSpec((1, INTER, DIM), wmap),
            pl.BlockSpec((1, INTER, DIM), wmap),
            pl.BlockSpec((1, DIM, INTER), wmap),
        ],
        out_specs=pl.BlockSpec((BS, DIM), lambda i, be_ref: (i, 0)),
    )
    return pl.pallas_call(
        _gmm_body,
        grid_spec=grid_spec,
        out_shape=jax.ShapeDtypeStruct((NPAD, DIM), jnp.float32),
        interpret=_INTERPRET,
    )(be, Xs, Ws, w1, w3, w2)


def _shared_body(x_ref, ws1_ref, ws3_ref, ws2_ref, g_ref, y_ref):
    xb = x_ref[...]
    h1 = lax.dot_general(xb, ws1_ref[...], (((1,), (1,)), ((), ())),
                         preferred_element_type=jnp.float32)
    h3 = lax.dot_general(xb, ws3_ref[...], (((1,), (1,)), ((), ())),
                         preferred_element_type=jnp.float32)
    h = h1 * jax.nn.sigmoid(h1) * h3
    z = lax.dot_general(h, ws2_ref[...], (((1,), (1,)), ((), ())),
                        preferred_element_type=jnp.float32)
    y_ref[...] = z + g_ref[...]


def _shared(x, ws1, ws3, ws2, g):
    return pl.pallas_call(
        _shared_body,
        grid=(T // BS,),
        in_specs=[
            pl.BlockSpec((BS, DIM), lambda i: (i, 0)),
            pl.BlockSpec((SI, DIM), lambda i: (0, 0)),
            pl.BlockSpec((SI, DIM), lambda i: (0, 0)),
            pl.BlockSpec((DIM, SI), lambda i: (0, 0)),
            pl.BlockSpec((BS, DIM), lambda i: (i, 0)),
        ],
        out_specs=pl.BlockSpec((BS, DIM), lambda i: (i, 0)),
        out_shape=jax.ShapeDtypeStruct((T, DIM), jnp.float32),
        interpret=_INTERPRET,
    )(x, ws1, ws3, ws2, g)


_NW = 32
_TPW = T // _NW


def _dispatch(xf, p0, p1, w0s, w1s):
    mesh = plsc.VectorSubcoreMesh(core_axis_name="c", subcore_axis_name="s")

    @functools.partial(
        pl.kernel, mesh=mesh,
        out_type=[
            jax.ShapeDtypeStruct((NPAD, DIM), jnp.float32),
            jax.ShapeDtypeStruct((NPAD, WL), jnp.float32),
        ],
        scratch_types=[
            pltpu.VMEM((_TPW,), jnp.int32),
            pltpu.VMEM((_TPW,), jnp.int32),
            pltpu.VMEM((_TPW, DIM), jnp.float32),
            pltpu.VMEM((_TPW, WL), jnp.float32),
            pltpu.VMEM((_TPW, WL), jnp.float32),
            pltpu.SemaphoreType.DMA,
        ],
    )
    def k(x_hbm, p0_hbm, p1_hbm, w0_hbm, w1_hbm, xs_hbm, ws_hbm,
          i0, i1, xa, wa0, wa1, sem):
        wid = lax.axis_index("s") * 2 + lax.axis_index("c")
        base = wid * _TPW
        pltpu.sync_copy(p0_hbm.at[pl.ds(base, _TPW)], i0)
        pltpu.sync_copy(p1_hbm.at[pl.ds(base, _TPW)], i1)
        pltpu.sync_copy(x_hbm.at[pl.ds(base, _TPW)], xa)
        pltpu.sync_copy(w0_hbm.at[pl.ds(base, _TPW)], wa0)
        pltpu.sync_copy(w1_hbm.at[pl.ds(base, _TPW)], wa1)
        c0 = pltpu.async_copy(xa, xs_hbm.at[i0], sem)
        c1 = pltpu.async_copy(xa, xs_hbm.at[i1], sem)
        c2 = pltpu.async_copy(wa0, ws_hbm.at[i0], sem)
        c3 = pltpu.async_copy(wa1, ws_hbm.at[i1], sem)
        c0.wait()
        c1.wait()
        c2.wait()
        c3.wait()

    return k(xf, p0, p1, w0s, w1s)


_CC = 16


def _combine(Yw, p0, p1):
    mesh = plsc.VectorSubcoreMesh(core_axis_name="c", subcore_axis_name="s")
    nj = _TPW // _CC

    @functools.partial(
        pl.kernel, mesh=mesh,
        out_type=jax.ShapeDtypeStruct((T, DIM), jnp.float32),
        scratch_types=[
            pltpu.VMEM((_TPW,), jnp.int32),
            pltpu.VMEM((_TPW,), jnp.int32),
            pltpu.VMEM((_CC, DIM), jnp.float32),
            pltpu.VMEM((_CC, DIM), jnp.float32),
            pltpu.VMEM((_CC, DIM), jnp.float32),
            pltpu.VMEM((_CC, DIM), jnp.float32),
            pltpu.SemaphoreType.DMA,
            pltpu.SemaphoreType.DMA,
        ],
    )
    def k(yw_hbm, p0_hbm, p1_hbm, g_hbm, i0, i1, a0, b0, a1, b1, s0, s1):
        wid = lax.axis_index("s") * 2 + lax.axis_index("c")
        base = wid * _TPW
        pltpu.sync_copy(p0_hbm.at[pl.ds(base, _TPW)], i0)
        pltpu.sync_copy(p1_hbm.at[pl.ds(base, _TPW)], i1)
        ab = [(a0, b0), (a1, b1)]
        sems = [s0, s1]

        def fetch(j):
            a, b = ab[j % 2]
            s = sems[j % 2]
            ia = i0[pl.ds(j * _CC, _CC)]
            ib = i1[pl.ds(j * _CC, _CC)]
            return (pltpu.async_copy(yw_hbm.at[ia], a, s),
                    pltpu.async_copy(yw_hbm.at[ib], b, s))

        pend = fetch(0)
        for j in range(nj):
            a, b = ab[j % 2]
            pend[0].wait()
            pend[1].wait()
            if j + 1 < nj:
                pend = fetch(j + 1)
            for t in range(_CC):
                def inner(c, _):
                    for u in range(4):
                        sl = pl.ds(c * 64 + u * 16, 16)
                        a[t, sl] = a[t, sl] + b[t, sl]
                    return 0
                lax.fori_loop(0, DIM // 64, inner, 0)
            pltpu.sync_copy(a, g_hbm.at[pl.ds(base + j * _CC, _CC)])

    return k(Yw, p0, p1)


def kernel(x, Wg, w1, w2, w3, ws1, ws2, ws3):
    shape = x.shape
    xf = x.reshape(T, DIM)
    pos0, pos1, w0s, w1s, be = _routing(xf, Wg)
    p0 = pos0.reshape(T)
    p1 = pos1.reshape(T)
    be = be.reshape(NB)

    Xs, Ws = _dispatch(xf, p0, p1, w0s, w1s)
    Yw = _gmm(be, Xs, Ws, w1, w3, w2)
    g = _combine(Yw, p0, p1)
    y = _shared(xf, ws1, ws3, ws2, g)
    return y.reshape(shape)
